# baseline (speedup 1.0000x reference)
import functools

import jax
import jax.numpy as jnp
from jax import lax
from jax.experimental import pallas as pl
from jax.experimental.pallas import tpu as pltpu
from jax.experimental.pallas import tpu_sc as plsc

EPS = 1e-06
NEG_INF = float("-inf")

B, S, D, E, K = 2, 2048, 1024, 8, 2
N = B * S
A = N * K
RT = 1024
NR = N // RT
TG = 256
NP = A + E * TG
NT = NP // TG
NW = 32
TPW = N // NW
CH = 64


def _top2(x_ref, wr_ref, br_ref):
    xb = x_ref[...].astype(jnp.bfloat16)
    logits = lax.dot_general(
        xb, wr_ref[...], (((1,), (1,)), ((), ())),
        preferred_element_type=jnp.float32) + br_ref[...]
    eidx = lax.broadcasted_iota(jnp.int32, (RT, E), 1)
    v0 = jnp.max(logits, axis=-1, keepdims=True)
    i0 = jnp.argmax(logits, axis=-1)
    masked = jnp.where(eidx == i0[:, None], NEG_INF, logits)
    v1 = jnp.max(masked, axis=-1, keepdims=True)
    i1 = jnp.argmax(masked, axis=-1)
    oh0 = (eidx == i0[:, None]).astype(jnp.float32)
    oh1 = (eidx == i1[:, None]).astype(jnp.float32)
    return oh0, oh1, v0, v1


def _count_kernel(x_ref, wr_ref, br_ref, cnt_ref):
    r = pl.program_id(0)
    oh0, oh1, _, _ = _top2(x_ref, wr_ref, br_ref)
    cnt_ref[pl.ds(0, 1), pl.ds(0, 1), :] = jnp.sum(oh0, axis=0, keepdims=True)[None]
    cnt_ref[pl.ds(0, 1), pl.ds(1, 1), :] = jnp.sum(oh1, axis=0, keepdims=True)[None]


def _slots_kernel(lt_ref, cnt_ref, x_ref, wr_ref, br_ref,
                  slot0_ref, slot1_ref, w0_ref, w1_ref, te_ref):
    r = pl.program_id(0)
    oh0, oh1, v0, v1 = _top2(x_ref, wr_ref, br_ref)

    full = cnt_ref[...]
    cnt0 = full[:, 0, :]
    cnt1 = full[:, 1, :]
    total0 = jnp.sum(cnt0, axis=0, keepdims=True)
    total = total0 + jnp.sum(cnt1, axis=0, keepdims=True)
    padded = jnp.ceil(total * (1.0 / TG)) * TG
    srcm = (lax.broadcasted_iota(jnp.int32, (E, E), 0)
            < lax.broadcasted_iota(jnp.int32, (E, E), 1)).astype(jnp.float32)
    base = lax.dot_general(
        padded, srcm, (((1,), (0,)), ((), ())),
        precision=jax.lax.Precision.HIGHEST,
        preferred_element_type=jnp.float32)
    rmask = (lax.broadcasted_iota(jnp.int32, (NR, E), 0)
             < r).astype(jnp.float32)
    pre0 = jnp.sum(cnt0 * rmask, axis=0, keepdims=True)
    pre1 = jnp.sum(cnt1 * rmask, axis=0, keepdims=True)
    g0 = base + pre0
    g1 = base + total0 + pre1

    ltm = lt_ref[...]
    cums0 = lax.dot_general(
        ltm, oh0.astype(jnp.bfloat16), (((1,), (0,)), ((), ())),
        preferred_element_type=jnp.float32)
    cums1 = lax.dot_general(
        ltm, oh1.astype(jnp.bfloat16), (((1,), (0,)), ((), ())),
        preferred_element_type=jnp.float32)

    slot0 = jnp.sum(oh0 * (g0 + cums0), axis=1, keepdims=True)
    slot1 = jnp.sum(oh1 * (g1 + cums1), axis=1, keepdims=True)
    slot0_ref[...] = slot0.astype(jnp.int32)
    slot1_ref[...] = slot1.astype(jnp.int32)

    bexp = jnp.exp(v1 - v0)
    denom = 1.0 + bexp
    w0_ref[...] = 1.0 / denom
    w1_ref[...] = bexp / denom

    bounds = jnp.transpose(base + padded)
    tstart = lax.broadcasted_iota(
        jnp.int32, (E, NT), 1).astype(jnp.float32) * float(TG)
    te = jnp.sum((bounds <= tstart).astype(jnp.int32),
                 axis=0, keepdims=True)
    te_ref[...] = jnp.minimum(te, E - 1)


def _router(xf, wr_bf, br, lt):
    br2 = br.reshape(1, E)
    cnt = pl.pallas_call(
        _count_kernel,
        grid=(NR,),
        in_specs=[
            pl.BlockSpec((RT, D), lambda r: (r, 0)),
            pl.BlockSpec((E, D), lambda r: (0, 0)),
            pl.BlockSpec((1, E), lambda r: (0, 0)),
        ],
        out_specs=pl.BlockSpec((1, 2, E), lambda r: (r, 0, 0)),
        out_shape=jax.ShapeDtypeStruct((NR, 2, E), jnp.float32),
        compiler_params=pltpu.CompilerParams(
            dimension_semantics=("arbitrary",)),
    )(xf, wr_bf, br2)
    return pl.pallas_call(
        _slots_kernel,
        grid=(NR,),
        in_specs=[
            pl.BlockSpec((RT, RT), lambda r: (0, 0)),
            pl.BlockSpec((NR, 2, E), lambda r: (0, 0, 0)),
            pl.BlockSpec((RT, D), lambda r: (r, 0)),
            pl.BlockSpec((E, D), lambda r: (0, 0)),
            pl.BlockSpec((1, E), lambda r: (0, 0)),
        ],
        out_specs=[
            pl.BlockSpec((RT, 1), lambda r: (r, 0)),
            pl.BlockSpec((RT, 1), lambda r: (r, 0)),
            pl.BlockSpec((RT, 1), lambda r: (r, 0)),
            pl.BlockSpec((RT, 1), lambda r: (r, 0)),
            pl.BlockSpec((1, NT), lambda r: (0, 0)),
        ],
        out_shape=[
            jax.ShapeDtypeStruct((N, 1), jnp.int32),
            jax.ShapeDtypeStruct((N, 1), jnp.int32),
            jax.ShapeDtypeStruct((N, 1), jnp.float32),
            jax.ShapeDtypeStruct((N, 1), jnp.float32),
            jax.ShapeDtypeStruct((1, NT), jnp.int32),
        ],
        compiler_params=pltpu.CompilerParams(
            dimension_semantics=("arbitrary",)),
    )(lt, cnt, xf, wr_bf, br2)


def _sc_dispatch_kernel(xb_hbm, s0_hbm, s1_hbm, xs_hbm,
                        s0_v, s1_v, rows_v, sem0, sem1):
    wid = lax.axis_index("s") * 2 + lax.axis_index("c")
    base = wid * TPW
    pltpu.sync_copy(s0_hbm.at[pl.ds(base, TPW)], s0_v)
    pltpu.sync_copy(s1_hbm.at[pl.ds(base, TPW)], s1_v)
    pltpu.sync_copy(xb_hbm.at[pl.ds(base, TPW)], rows_v)
    cp0 = pltpu.async_copy(rows_v, xs_hbm.at[s0_v], sem0)
    cp1 = pltpu.async_copy(rows_v, xs_hbm.at[s1_v], sem1)
    cp0.wait()
    cp1.wait()


def _dispatch(xb32, s0, s1):
    mesh = plsc.VectorSubcoreMesh(core_axis_name="c", subcore_axis_name="s")
    f = functools.partial(
        pl.kernel, mesh=mesh,
        out_type=jax.ShapeDtypeStruct((NP, D // 2), jnp.int32),
        scratch_types=[
            pltpu.VMEM((TPW,), jnp.int32),
            pltpu.VMEM((TPW,), jnp.int32),
            pltpu.VMEM((TPW, D // 2), jnp.int32),
            pltpu.SemaphoreType.DMA,
            pltpu.SemaphoreType.DMA,
        ],
        compiler_params=pltpu.CompilerParams(use_tc_tiling_on_sc=True),
        )(_sc_dispatch_kernel)
    return f(xb32, s0, s1)


def _gmm_kernel(te_ref, xs_ref, we_ref, be_ref, ys_ref):
    i = pl.program_id(0)
    e = te_ref[i]
    h = lax.dot_general(
        xs_ref[...], we_ref[e], (((1,), (1,)), ((), ())),
        preferred_element_type=jnp.float32)
    h = h + be_ref[pl.ds(e, 1), :]
    ys_ref[...] = h * jax.nn.sigmoid(h)


def _gmm(te, xs, we_bf, be):
    grid_spec = pltpu.PrefetchScalarGridSpec(
        num_scalar_prefetch=1,
        grid=(NT,),
        in_specs=[
            pl.BlockSpec((TG, D), lambda i, te: (i, 0)),
            pl.BlockSpec((E, D, D), lambda i, te: (0, 0, 0)),
            pl.BlockSpec((E, D), lambda i, te: (0, 0)),
        ],
        out_specs=pl.BlockSpec((TG, D), lambda i, te: (i, 0)),
    )
    return pl.pallas_call(
        _gmm_kernel,
        grid_spec=grid_spec,
        out_shape=jax.ShapeDtypeStruct((NP, D), jnp.float32),
        compiler_params=pltpu.CompilerParams(
            dimension_semantics=("arbitrary",),
            vmem_limit_bytes=100 * 1024 * 1024,
        ),
    )(te, xs, we_bf, be)


def _sc_combine_kernel(ys_hbm, s0_hbm, s1_hbm, sel0_hbm, sel1_hbm,
                       idx_v, rows_v, sem):
    wid = lax.axis_index("s") * 2 + lax.axis_index("c")
    base = wid * TPW
    for s_hbm, sel_hbm in ((s0_hbm, sel0_hbm), (s1_hbm, sel1_hbm)):
        for c in range(TPW // CH):
            off = base + c * CH
            pltpu.sync_copy(s_hbm.at[pl.ds(off, CH)], idx_v)
            pltpu.async_copy(ys_hbm.at[idx_v], rows_v, sem).wait()
            pltpu.sync_copy(rows_v, sel_hbm.at[pl.ds(off, CH)])


def _combine(ys, s0, s1):
    mesh = plsc.VectorSubcoreMesh(core_axis_name="c", subcore_axis_name="s")
    f = functools.partial(
        pl.kernel, mesh=mesh,
        out_type=(jax.ShapeDtypeStruct((N, D), jnp.float32),
                  jax.ShapeDtypeStruct((N, D), jnp.float32)),
        scratch_types=[
            pltpu.VMEM((CH,), jnp.int32),
            pltpu.VMEM((CH, D), jnp.float32),
            pltpu.SemaphoreType.DMA,
        ],
        compiler_params=pltpu.CompilerParams(use_tc_tiling_on_sc=True),
        )(_sc_combine_kernel)
    return f(ys, s0, s1)


FT = 512


def _final_kernel(x_ref, w0_ref, w1_ref, sel0_ref, sel1_ref, wo_ref, bo_ref,
                  nw_ref, out_ref):
    xt = x_ref[...]
    comb = w0_ref[...] * sel0_ref[...] + w1_ref[...] * sel1_ref[...]
    out = lax.dot_general(
        comb.astype(jnp.bfloat16), wo_ref[...], (((1,), (1,)), ((), ())),
        preferred_element_type=jnp.float32) + bo_ref[...]
    res = xt + out
    rms = jnp.sqrt(jnp.mean(res * res, axis=-1, keepdims=True) + EPS)
    out_ref[...] = nw_ref[...] * (res / rms)


def _finalize(xf, w0, w1, sel0, sel1, wo_bf, bo, norm_w):
    return pl.pallas_call(
        _final_kernel,
        grid=(N // FT,),
        in_specs=[
            pl.BlockSpec((FT, D), lambda i: (i, 0)),
            pl.BlockSpec((FT, 1), lambda i: (i, 0)),
            pl.BlockSpec((FT, 1), lambda i: (i, 0)),
            pl.BlockSpec((FT, D), lambda i: (i, 0)),
            pl.BlockSpec((FT, D), lambda i: (i, 0)),
            pl.BlockSpec((D, D), lambda i: (0, 0)),
            pl.BlockSpec((1, D), lambda i: (0, 0)),
            pl.BlockSpec((1, D), lambda i: (0, 0)),
        ],
        out_specs=pl.BlockSpec((FT, D), lambda i: (i, 0)),
        out_shape=jax.ShapeDtypeStruct((N, D), jnp.float32),
    )(xf, w0, w1, sel0, sel1, wo_bf, bo.reshape(1, D), norm_w.reshape(1, D))


@jax.jit
def kernel(x, Wr, br, We, be, Wo, bo, norm_w):
    xf = x.reshape(N, D)
    xb = xf.astype(jnp.bfloat16)
    wr_bf = Wr.astype(jnp.bfloat16)
    we_bf = We.astype(jnp.bfloat16)
    wo_bf = Wo.astype(jnp.bfloat16)
    lt = jnp.tril(jnp.ones((RT, RT), jnp.bfloat16), k=-1)

    slot0, slot1, w0, w1, te = _router(xf, wr_bf, br, lt)
    s0 = slot0.reshape(N)
    s1 = slot1.reshape(N)
    xb32 = lax.bitcast_convert_type(xb.reshape(N, D // 2, 2), jnp.int32)
    xs32 = _dispatch(xb32, s0, s1)
    xs = lax.bitcast_convert_type(xs32, jnp.bfloat16).reshape(NP, D)
    ys = _gmm(te.reshape(NT), xs, we_bf, be)
    sel0, sel1 = _combine(ys, s0, s1)
    out = _finalize(xf, w0, w1, sel0, sel1, wo_bf, bo, norm_w)
    return out.reshape(B, S, D)


# speedup vs baseline: 3.8170x; 3.8170x over previous
import functools

import jax
import jax.numpy as jnp
from jax.experimental import pallas as pl
from jax.experimental.pallas import tpu as pltpu

EPS = 1e-06
NEG_INF = float("-inf")


def _fused_kernel(x_ref, wr_ref, br_ref, we_ref, be_ref, wo_ref, bo_ref,
                  nw_ref, out_ref):
    xt = x_ref[...]
    E = wr_ref.shape[0]
    T = xt.shape[0]

    xb = xt.astype(jnp.bfloat16)
    logits = jax.lax.dot_general(
        xb, wr_ref[...].astype(jnp.bfloat16), (((1,), (1,)), ((), ())),
        preferred_element_type=jnp.float32) + br_ref[...]

    eidx = jax.lax.broadcasted_iota(jnp.int32, (T, E), 1)
    v0 = jnp.max(logits, axis=-1, keepdims=True)
    i0 = jnp.argmax(logits, axis=-1)
    masked = jnp.where(eidx == i0[:, None], NEG_INF, logits)
    v1 = jnp.max(masked, axis=-1, keepdims=True)
    i1 = jnp.argmax(masked, axis=-1)

    b = jnp.exp(v1 - v0)
    denom = 1.0 + b
    w0 = (1.0 / denom)[:, 0]
    w1 = (b / denom)[:, 0]

    comb = jnp.zeros(xt.shape, jnp.bfloat16)
    for e in range(E):
        h = jax.lax.dot_general(
            xb, we_ref[e], (((1,), (1,)), ((), ())),
            preferred_element_type=jnp.float32)
        h = (h + be_ref[e][None, :]).astype(jnp.bfloat16)
        h = h * jax.nn.sigmoid(h)
        wt = jnp.where(i0 == e, w0, 0.0) + jnp.where(i1 == e, w1, 0.0)
        comb = comb + wt.astype(jnp.bfloat16)[:, None] * h

    out = jax.lax.dot_general(
        comb, wo_ref[...], (((1,), (1,)), ((), ())),
        preferred_element_type=jnp.float32) + bo_ref[...]
    res = xt + out
    rms = jnp.sqrt(jnp.mean(res * res, axis=-1, keepdims=True) + EPS)
    out_ref[...] = nw_ref[...] * (res / rms)


@jax.jit
def kernel(x, Wr, br, We, be, Wo, bo, norm_w):
    B, S, D = x.shape
    E = Wr.shape[0]
    N = B * S
    T = 512

    xf = x.reshape(N, D)
    we_bf = We.astype(jnp.bfloat16)
    wo_bf = Wo.astype(jnp.bfloat16)

    out = pl.pallas_call(
        _fused_kernel,
        grid=(N // T,),
        in_specs=[
            pl.BlockSpec((T, D), lambda i: (i, 0)),
            pl.BlockSpec((E, D), lambda i: (0, 0)),
            pl.BlockSpec((1, E), lambda i: (0, 0)),
            pl.BlockSpec((E, D, D), lambda i: (0, 0, 0)),
            pl.BlockSpec((E, D), lambda i: (0, 0)),
            pl.BlockSpec((D, D), lambda i: (0, 0)),
            pl.BlockSpec((1, D), lambda i: (0, 0)),
            pl.BlockSpec((1, D), lambda i: (0, 0)),
        ],
        out_specs=pl.BlockSpec((T, D), lambda i: (i, 0)),
        out_shape=jax.ShapeDtypeStruct((N, D), jnp.float32),
        compiler_params=pltpu.CompilerParams(
            dimension_semantics=("arbitrary",),
            vmem_limit_bytes=100 * 1024 * 1024,
        ),
    )(xf, Wr, br.reshape(1, E), we_bf, be, wo_bf, bo.reshape(1, D),
      norm_w.reshape(1, D))
    return out.reshape(B, S, D)


# speedup vs baseline: 3.9434x; 1.0331x over previous
import functools

import jax
import jax.numpy as jnp
from jax.experimental import pallas as pl
from jax.experimental.pallas import tpu as pltpu

EPS = 1e-06
NEG_INF = float("-inf")


def _fused_kernel(x_ref, wr_ref, br_ref, we_ref, be_ref, wo_ref, bo_ref,
                  nw_ref, out_ref):
    xt = x_ref[...]
    E = wr_ref.shape[0]
    T = xt.shape[0]

    xb = xt.astype(jnp.bfloat16)
    logits = jax.lax.dot_general(
        xb, wr_ref[...].astype(jnp.bfloat16), (((1,), (1,)), ((), ())),
        preferred_element_type=jnp.float32) + br_ref[...]

    eidx = jax.lax.broadcasted_iota(jnp.int32, (T, E), 1)
    v0 = jnp.max(logits, axis=-1, keepdims=True)
    i0 = jnp.argmax(logits, axis=-1)
    masked = jnp.where(eidx == i0[:, None], NEG_INF, logits)
    v1 = jnp.max(masked, axis=-1, keepdims=True)
    i1 = jnp.argmax(masked, axis=-1)

    b = jnp.exp(v1 - v0)
    denom = 1.0 + b
    w0 = (1.0 / denom)[:, 0]
    w1 = (b / denom)[:, 0]

    comb = jnp.zeros(xt.shape, jnp.bfloat16)
    for e in range(E):
        h = jax.lax.dot_general(
            xb, we_ref[e], (((1,), (1,)), ((), ())),
            preferred_element_type=jnp.float32)
        h = (h + be_ref[e][None, :]).astype(jnp.bfloat16)
        h = h * jax.nn.sigmoid(h)
        wt = jnp.where(i0 == e, w0, 0.0) + jnp.where(i1 == e, w1, 0.0)
        comb = comb + wt.astype(jnp.bfloat16)[:, None] * h

    out = jax.lax.dot_general(
        comb, wo_ref[...], (((1,), (1,)), ((), ())),
        preferred_element_type=jnp.float32) + bo_ref[...]
    res = xt + out
    rms = jnp.sqrt(jnp.mean(res * res, axis=-1, keepdims=True) + EPS)
    out_ref[...] = nw_ref[...] * (res / rms)


@jax.jit
def kernel(x, Wr, br, We, be, Wo, bo, norm_w):
    B, S, D = x.shape
    E = Wr.shape[0]
    N = B * S
    T = 1024

    xf = x.reshape(N, D)
    we_bf = We.astype(jnp.bfloat16)
    wo_bf = Wo.astype(jnp.bfloat16)

    out = pl.pallas_call(
        _fused_kernel,
        grid=(N // T,),
        in_specs=[
            pl.BlockSpec((T, D), lambda i: (i, 0)),
            pl.BlockSpec((E, D), lambda i: (0, 0)),
            pl.BlockSpec((1, E), lambda i: (0, 0)),
            pl.BlockSpec((E, D, D), lambda i: (0, 0, 0)),
            pl.BlockSpec((E, D), lambda i: (0, 0)),
            pl.BlockSpec((D, D), lambda i: (0, 0)),
            pl.BlockSpec((1, D), lambda i: (0, 0)),
            pl.BlockSpec((1, D), lambda i: (0, 0)),
        ],
        out_specs=pl.BlockSpec((T, D), lambda i: (i, 0)),
        out_shape=jax.ShapeDtypeStruct((N, D), jnp.float32),
        compiler_params=pltpu.CompilerParams(
            dimension_semantics=("arbitrary",),
            vmem_limit_bytes=100 * 1024 * 1024,
        ),
    )(xf, Wr, br.reshape(1, E), we_bf, be, wo_bf, bo.reshape(1, D),
      norm_w.reshape(1, D))
    return out.reshape(B, S, D)
